# baseline (speedup 1.0000x reference)
import jax
import jax.numpy as jnp
from jax.experimental import pallas as pl
from jax.experimental.pallas import tpu as pltpu

B, C, H, W = 64, 768, 24, 24
HW = H * W
E = 8
BB = 16
NSTEPS = B // BB
NEG_INF = float("-inf")


def _gate_kernel(x_ref, w0_ref, b0_ref, w1_ref, b1_ref, out_ref, acc):
    j = pl.program_id(0)
    blk = x_ref[...]
    acc[pl.ds(j * BB, BB), :] = (jnp.max(blk, axis=1)
                                 + jnp.sum(blk, axis=1) * (1.0 / HW))

    @pl.when(j == NSTEPS - 1)
    def _epilogue():
        pooled = acc[...]
        h = jnp.dot(pooled, w0_ref[...],
                    preferred_element_type=jnp.float32) + b0_ref[...]
        h = jnp.where(h >= 0.0, h, 0.2 * h)
        z = jnp.dot(pooled, w1_ref[...],
                    preferred_element_type=jnp.float32) + b1_ref[...]
        noise = jnp.maximum(z, 0.0) + jnp.log1p(jnp.exp(-jnp.abs(z)))
        nmean = jnp.mean(noise, axis=1, keepdims=True)
        var = jnp.sum((noise - nmean) ** 2, axis=1, keepdims=True) / (E - 1)
        norm_noise = (noise - nmean) * jax.lax.rsqrt(var)
        scores = h + norm_noise
        ii = jax.lax.broadcasted_iota(jnp.int32, (B, E), 1)
        m1 = jnp.max(scores, axis=1, keepdims=True)
        i1 = jnp.min(jnp.where(scores == m1, ii, E), axis=1, keepdims=True)
        oh1 = ii == i1
        s2 = jnp.where(oh1, NEG_INF, scores)
        m2 = jnp.max(s2, axis=1, keepdims=True)
        i2 = jnp.min(jnp.where(s2 == m2, ii, E), axis=1, keepdims=True)
        mask = oh1 | (ii == i2)
        hm = jnp.where(mask, h, NEG_INF)
        mx = jnp.max(hm, axis=1, keepdims=True)
        e = jnp.where(mask, jnp.exp(h - mx), 0.0)
        gate = e / jnp.sum(e, axis=1, keepdims=True)
        out_ref[...] = gate.T


@jax.jit
def kernel(x, W0, b0, W1, b1):
    xt = jnp.transpose(x, (0, 2, 3, 1)).reshape(B, HW, C)
    gate_t = pl.pallas_call(
        _gate_kernel,
        grid=(NSTEPS,),
        in_specs=[
            pl.BlockSpec((BB, HW, C), lambda j: (j, 0, 0)),
            pl.BlockSpec((C, E), lambda j: (0, 0)),
            pl.BlockSpec((1, E), lambda j: (0, 0)),
            pl.BlockSpec((C, E), lambda j: (0, 0)),
            pl.BlockSpec((1, E), lambda j: (0, 0)),
        ],
        out_specs=pl.BlockSpec((E, B), lambda j: (0, 0)),
        out_shape=jax.ShapeDtypeStruct((E, B), jnp.float32),
        scratch_shapes=[pltpu.VMEM((B, C), jnp.float32)],
    )(xt, W0.T, b0.reshape(1, E), W1.T, b1.reshape(1, E))
    return gate_t.T
